# Initial kernel scaffold; baseline (speedup 1.0000x reference)
import jax
import jax.numpy as jnp
from jax.experimental import pallas as pl


def kernel(nodes, neigh_idx, perm, features, W, alpha):
    raise NotImplementedError("write your pallas kernel here")


# speedup vs baseline: 1.3445x; 1.3445x over previous
import functools

import jax
import jax.numpy as jnp
from jax import lax
from jax.experimental import pallas as pl
from jax.experimental.pallas import tpu as pltpu
from jax.experimental.pallas import tpu_sc as plsc

N_NODES = 100000
D = 128
B = 10000
S = 32
FAN = S + 1

NC = 2
NS = 16
NW = NC * NS

RW = 320
BP = NW * RW
ROW_STRIDE = 40
IDX_CHUNK = 128
KW = 12928
IDX_CHUNKS = KW // IDX_CHUNK

_INV_FAN = 1.0 / float(FAN)


def _fire(feat_hbm, idx1_v, idx2_v, row, b1, b2, sem):
    s = row * ROW_STRIDE
    pltpu.async_copy(feat_hbm.at[idx1_v.at[pl.ds(s, FAN)]], b1, sem)
    pltpu.async_copy(feat_hbm.at[idx2_v.at[pl.ds(s, FAN)]], b2, sem)


def _drain(feat_hbm, idx1_v, b1, b2, sem):
    pltpu.make_async_copy(feat_hbm.at[idx1_v.at[pl.ds(0, FAN)]], b1, sem).wait()
    pltpu.make_async_copy(feat_hbm.at[idx1_v.at[pl.ds(0, FAN)]], b2, sem).wait()


def _accum(buf, out_ref, row):
    inv = jnp.float32(_INV_FAN)
    for c in range(D // 16):
        d = pl.ds(16 * c, 16)
        s0 = buf[0, d]
        s1 = buf[1, d]
        s2 = buf[2, d]
        s3 = buf[3, d]
        for j in range(4, 32, 4):
            s0 = s0 + buf[j, d]
            s1 = s1 + buf[j + 1, d]
            s2 = s2 + buf[j + 2, d]
            s3 = s3 + buf[j + 3, d]
        s0 = s0 + buf[32, d]
        out_ref[row, d] = ((s0 + s1) + (s2 + s3)) * inv


def _sc_body(idx_hbm, perm_hbm, feat_hbm, out1_hbm, out2_hbm,
             idx1_v, idx2_v, buf1, buf2, o1, o2, sem_a, sem_b, sem_i):
    wid = lax.axis_index("s") * NC + lax.axis_index("c")

    pltpu.sync_copy(idx_hbm.at[pl.ds(wid * KW, KW)], idx1_v)

    K = 8
    for c in range(IDX_CHUNKS):
        pltpu.async_copy(
            perm_hbm.at[idx1_v.at[pl.ds(c * IDX_CHUNK, IDX_CHUNK)]],
            idx2_v.at[pl.ds(c * IDX_CHUNK, IDX_CHUNK)], sem_i)
        if c >= K:
            pltpu.make_async_copy(
                perm_hbm.at[pl.ds(0, IDX_CHUNK)],
                idx2_v.at[pl.ds((c - K) * IDX_CHUNK, IDX_CHUNK)], sem_i).wait()
    for c in range(IDX_CHUNKS - K, IDX_CHUNKS):
        pltpu.make_async_copy(
            perm_hbm.at[pl.ds(0, IDX_CHUNK)],
            idx2_v.at[pl.ds(c * IDX_CHUNK, IDX_CHUNK)], sem_i).wait()

    _fire(feat_hbm, idx1_v, idx2_v, 0, buf1.at[0], buf2.at[0], sem_a)

    def body(t, carry):
        r0 = 2 * t
        _fire(feat_hbm, idx1_v, idx2_v, r0 + 1, buf1.at[1], buf2.at[1], sem_b)
        _drain(feat_hbm, idx1_v, buf1.at[0], buf2.at[0], sem_a)
        _accum(buf1.at[0], o1, r0)
        _accum(buf2.at[0], o2, r0)
        _fire(feat_hbm, idx1_v, idx2_v, r0 + 2, buf1.at[0], buf2.at[0], sem_a)
        _drain(feat_hbm, idx1_v, buf1.at[1], buf2.at[1], sem_b)
        _accum(buf1.at[1], o1, r0 + 1)
        _accum(buf2.at[1], o2, r0 + 1)
        return carry

    lax.fori_loop(0, RW // 2, body, 0)
    _drain(feat_hbm, idx1_v, buf1.at[0], buf2.at[0], sem_a)

    pltpu.sync_copy(o1, out1_hbm.at[pl.ds(wid * RW, RW)])
    pltpu.sync_copy(o2, out2_hbm.at[pl.ds(wid * RW, RW)])


_sc_aggregate = functools.partial(
    pl.kernel,
    mesh=plsc.VectorSubcoreMesh(core_axis_name="c", subcore_axis_name="s"),
    out_type=[jax.ShapeDtypeStruct((BP, D), jnp.float32),
              jax.ShapeDtypeStruct((BP, D), jnp.float32)],
    scratch_types=[
        pltpu.VMEM((KW,), jnp.int32),
        pltpu.VMEM((KW,), jnp.int32),
        pltpu.VMEM((2, FAN, D), jnp.float32),
        pltpu.VMEM((2, FAN, D), jnp.float32),
        pltpu.VMEM((RW, D), jnp.float32),
        pltpu.VMEM((RW, D), jnp.float32),
        pltpu.SemaphoreType.DMA,
        pltpu.SemaphoreType.DMA,
        pltpu.SemaphoreType.DMA,
    ],
)(_sc_body)


TC_BLK = 512


def _tc_body(a1_ref, a2_ref, w_ref, alpha_ref, o1_ref, o2_ref):
    w = w_ref[...]
    al = alpha_ref[0, 0]
    dn = (((1,), (1,)), ((), ()))
    y1 = lax.dot_general(w, a1_ref[...], dn,
                         preferred_element_type=jnp.float32,
                         precision=lax.Precision.HIGHEST)
    o1_ref[...] = jnp.where(y1 >= 0, y1, al * y1)
    y2 = lax.dot_general(w, a2_ref[...], dn,
                         preferred_element_type=jnp.float32,
                         precision=lax.Precision.HIGHEST)
    o2_ref[...] = jnp.where(y2 >= 0, y2, al * y2)


def _tc_combine(agg1, agg2, W, alpha2d):
    return pl.pallas_call(
        _tc_body,
        grid=(BP // TC_BLK,),
        in_specs=[
            pl.BlockSpec((TC_BLK, D), lambda i: (i, 0)),
            pl.BlockSpec((TC_BLK, D), lambda i: (i, 0)),
            pl.BlockSpec((D, D), lambda i: (0, 0)),
            pl.BlockSpec(memory_space=pltpu.SMEM),
        ],
        out_specs=[
            pl.BlockSpec((D, TC_BLK), lambda i: (0, i)),
            pl.BlockSpec((D, TC_BLK), lambda i: (0, i)),
        ],
        out_shape=[jax.ShapeDtypeStruct((D, BP), jnp.float32),
                   jax.ShapeDtypeStruct((D, BP), jnp.float32)],
    )(agg1, agg2, W, alpha2d)


def kernel(nodes, neigh_idx, perm, features, W, alpha):
    idx1 = jnp.concatenate([neigh_idx, nodes[:, None]], axis=1)
    idxp = jnp.zeros((BP, ROW_STRIDE), jnp.int32).at[:B, :FAN].set(idx1)
    idxp = idxp.reshape(NW, RW * ROW_STRIDE)
    idx_hbm = (jnp.zeros((NW, KW), jnp.int32)
               .at[:, :RW * ROW_STRIDE].set(idxp).reshape(NW * KW))

    agg1, agg2 = _sc_aggregate(idx_hbm, perm, features)
    out1, out2 = _tc_combine(agg1, agg2, W, alpha.reshape(1, 1))
    return out1[:, :B], out2[:, :B]
